# Initial kernel scaffold; baseline (speedup 1.0000x reference)
import jax
import jax.numpy as jnp
from jax.experimental import pallas as pl


def kernel(indices, table):
    raise NotImplementedError("write your pallas kernel here")


# speedup vs baseline: 1.9056x; 1.9056x over previous
import functools

import jax
import jax.numpy as jnp
from jax import lax
from jax.experimental import pallas as pl
from jax.experimental.pallas import tpu as pltpu
from jax.experimental.pallas import tpu_sc as plsc

BATCH = 16384
NUM_FIELDS = 26
EMBED_DIM = 32
NTOT = BATCH * NUM_FIELDS
NUM_CORES = 2
NUM_SUBCORES = 16
NW = NUM_CORES * NUM_SUBCORES
PER_W = NTOT // NW
G = 128
NG = PER_W // G
GSZ = 4
GROW = G * GSZ
GROUPS = NG // GSZ

_mesh = plsc.VectorSubcoreMesh(core_axis_name="c", subcore_axis_name="s")


@functools.partial(
    pl.kernel,
    out_type=jax.ShapeDtypeStruct((NTOT, EMBED_DIM), jnp.float32),
    mesh=_mesh,
    scratch_types=[
        pltpu.VMEM((NG, G), jnp.int32),
        pltpu.VMEM((GROW, EMBED_DIM), jnp.float32),
        pltpu.VMEM((GROW, EMBED_DIM), jnp.float32),
        pltpu.SemaphoreType.DMA,
        pltpu.SemaphoreType.DMA,
    ],
    compiler_params=pltpu.CompilerParams(use_tc_tiling_on_sc=False),
)
def _sc_gather(idx_hbm, table_hbm, out_hbm, idx_v, buf0, buf1, sem0, sem1):
    wid = lax.axis_index("s") * NUM_CORES + lax.axis_index("c")
    base = wid * PER_W
    pltpu.sync_copy(idx_hbm.at[wid], idx_v)

    def fire(grp, buf, sem):
        for j in range(GSZ):
            pltpu.async_copy(
                table_hbm.at[idx_v.at[grp * GSZ + j]],
                buf.at[pl.ds(j * G, G)],
                sem,
            )

    def drain(buf, sem):
        pltpu.make_async_copy(table_hbm.at[pl.ds(0, GROW)], buf, sem).wait()

    fire(0, buf0, sem0)
    fire(1, buf1, sem1)

    @pl.loop(0, GROUPS, step=2)
    def _(g):
        for b, (buf, sem) in enumerate(((buf0, sem0), (buf1, sem1))):
            grp = g + b
            drain(buf, sem)
            pltpu.sync_copy(buf, out_hbm.at[pl.ds(base + grp * GROW, GROW)])
            nxt = grp + 2

            @pl.when(nxt < GROUPS)
            def _():
                fire(nxt, buf, sem)


def kernel(indices, table):
    idx = indices.reshape(NW, NG, G)
    out = _sc_gather(idx, table)
    return out.reshape(BATCH, NUM_FIELDS * EMBED_DIM)


# speedup vs baseline: 1.9085x; 1.0015x over previous
import functools

import jax
import jax.numpy as jnp
from jax import lax
from jax.experimental import pallas as pl
from jax.experimental.pallas import tpu as pltpu
from jax.experimental.pallas import tpu_sc as plsc

BATCH = 16384
NUM_FIELDS = 26
EMBED_DIM = 32
NTOT = BATCH * NUM_FIELDS
NUM_CORES = 2
NUM_SUBCORES = 16
NW = NUM_CORES * NUM_SUBCORES
PER_W = NTOT // NW
G = 128
NG = PER_W // G
GSZ = 8
GROW = G * GSZ
GROUPS = NG // GSZ
NBUF = 3

_mesh = plsc.VectorSubcoreMesh(core_axis_name="c", subcore_axis_name="s")


@functools.partial(
    pl.kernel,
    out_type=jax.ShapeDtypeStruct((NTOT, EMBED_DIM), jnp.float32),
    mesh=_mesh,
    scratch_types=[
        pltpu.VMEM((NG, G), jnp.int32),
        pltpu.VMEM((GROW, EMBED_DIM), jnp.float32),
        pltpu.VMEM((GROW, EMBED_DIM), jnp.float32),
        pltpu.VMEM((GROW, EMBED_DIM), jnp.float32),
        pltpu.SemaphoreType.DMA,
        pltpu.SemaphoreType.DMA,
        pltpu.SemaphoreType.DMA,
        pltpu.SemaphoreType.DMA,
        pltpu.SemaphoreType.DMA,
        pltpu.SemaphoreType.DMA,
    ],
    compiler_params=pltpu.CompilerParams(use_tc_tiling_on_sc=False),
)
def _sc_gather(idx_hbm, table_hbm, out_hbm, idx_v,
               buf0, buf1, buf2, g0, g1, g2, s0, s1, s2):
    wid = lax.axis_index("s") * NUM_CORES + lax.axis_index("c")
    base = wid * PER_W
    pltpu.sync_copy(idx_hbm.at[wid], idx_v)
    bufs = (buf0, buf1, buf2)
    gsems = (g0, g1, g2)
    ssems = (s0, s1, s2)

    def fire(grp, buf, gsem):
        for j in range(GSZ):
            pltpu.async_copy(
                table_hbm.at[idx_v.at[grp * GSZ + j]],
                buf.at[pl.ds(j * G, G)],
                gsem,
            )

    def drain_gather(buf, gsem):
        pltpu.make_async_copy(table_hbm.at[pl.ds(0, GROW)], buf, gsem).wait()

    def wait_store(buf, ssem):
        pltpu.make_async_copy(buf, out_hbm.at[pl.ds(0, GROW)], ssem).wait()

    for b in range(NBUF):
        fire(b, bufs[b], gsems[b])

    @pl.loop(0, GROUPS + (-GROUPS) % NBUF, step=NBUF)
    def _(g):
        for b in range(NBUF):
            grp = g + b

            @pl.when(grp < GROUPS)
            def _():
                drain_gather(bufs[b], gsems[b])
                pltpu.async_copy(
                    bufs[b],
                    out_hbm.at[pl.ds(base + grp * GROW, GROW)],
                    ssems[b],
                )
                nxt = grp + NBUF

                @pl.when(nxt < GROUPS)
                def _():
                    wait_store(bufs[b], ssems[b])
                    fire(nxt, bufs[b], gsems[b])

    for b in range(NBUF):
        wait_store(bufs[b], ssems[b])


def kernel(indices, table):
    idx = indices.reshape(NW, NG, G)
    out = _sc_gather(idx, table)
    return out.reshape(BATCH, NUM_FIELDS * EMBED_DIM)


# speedup vs baseline: 1.9161x; 1.0040x over previous
import functools

import jax
import jax.numpy as jnp
from jax import lax
from jax.experimental import pallas as pl
from jax.experimental.pallas import tpu as pltpu
from jax.experimental.pallas import tpu_sc as plsc

BATCH = 16384
NUM_FIELDS = 26
EMBED_DIM = 32
NTOT = BATCH * NUM_FIELDS
NUM_CORES = 2
NUM_SUBCORES = 16
NW = NUM_CORES * NUM_SUBCORES
PER_W = NTOT // NW
G = 128
NG = PER_W // G
GSZ = 8
GROW = G * GSZ
GROUPS = NG // GSZ
NBUF = 3

_mesh = plsc.VectorSubcoreMesh(core_axis_name="c", subcore_axis_name="s")


@functools.partial(
    pl.kernel,
    out_type=jax.ShapeDtypeStruct((NTOT, EMBED_DIM), jnp.float32),
    mesh=_mesh,
    scratch_types=[
        pltpu.VMEM((PER_W,), jnp.int32),
        pltpu.VMEM((GROW, EMBED_DIM), jnp.float32),
        pltpu.VMEM((GROW, EMBED_DIM), jnp.float32),
        pltpu.VMEM((GROW, EMBED_DIM), jnp.float32),
        pltpu.SemaphoreType.DMA,
        pltpu.SemaphoreType.DMA,
        pltpu.SemaphoreType.DMA,
        pltpu.SemaphoreType.DMA,
        pltpu.SemaphoreType.DMA,
        pltpu.SemaphoreType.DMA,
    ],
    compiler_params=pltpu.CompilerParams(use_tc_tiling_on_sc=False),
)
def _sc_gather(idx_hbm, table_hbm, out_hbm, idx_v,
               buf0, buf1, buf2, g0, g1, g2, s0, s1, s2):
    wid = lax.axis_index("s") * NUM_CORES + lax.axis_index("c")
    base = wid * PER_W
    pltpu.sync_copy(idx_hbm.at[pl.ds(wid * PER_W, PER_W)], idx_v)
    bufs = (buf0, buf1, buf2)
    gsems = (g0, g1, g2)
    ssems = (s0, s1, s2)

    def fire(grp, buf, gsem):
        for j in range(GSZ):
            pltpu.async_copy(
                table_hbm.at[idx_v.at[pl.ds((grp * GSZ + j) * G, G)]],
                buf.at[pl.ds(j * G, G)],
                gsem,
            )

    def drain_gather(buf, gsem):
        pltpu.make_async_copy(table_hbm.at[pl.ds(0, GROW)], buf, gsem).wait()

    def wait_store(buf, ssem):
        pltpu.make_async_copy(buf, out_hbm.at[pl.ds(0, GROW)], ssem).wait()

    for b in range(NBUF):
        fire(b, bufs[b], gsems[b])

    @pl.loop(0, GROUPS + (-GROUPS) % NBUF, step=NBUF)
    def _(g):
        for b in range(NBUF):
            grp = g + b

            @pl.when(grp < GROUPS)
            def _():
                drain_gather(bufs[b], gsems[b])
                pltpu.async_copy(
                    bufs[b],
                    out_hbm.at[pl.ds(base + grp * GROW, GROW)],
                    ssems[b],
                )
                nxt = grp + NBUF

                @pl.when(nxt < GROUPS)
                def _():
                    wait_store(bufs[b], ssems[b])
                    fire(nxt, bufs[b], gsems[b])

    for b in range(NBUF):
        wait_store(bufs[b], ssems[b])


def kernel(indices, table):
    idx = indices.reshape(NTOT)
    out = _sc_gather(idx, table)
    return out.reshape(BATCH, NUM_FIELDS * EMBED_DIM)
